# baseline (speedup 1.0000x reference)
import functools

import jax
import jax.numpy as jnp
from jax import lax
from jax.experimental import pallas as pl
from jax.experimental.pallas import tpu as pltpu
from jax.experimental.pallas import tpu_sc as plsc

SEQ = 200
EMB = 64
SCALE = 8.0

_info = plsc.get_sparse_core_info()
_NC, _NS, _NL = _info.num_cores, _info.num_subcores, _info.num_lanes
_NW = _NC * _NS


def _build(n_rows: int, chunk: int):
  assert n_rows % (_NW * chunk) == 0
  bpw = n_rows // _NW
  n_chunks = bpw // chunk
  assert bpw % SEQ == 0
  assert chunk % SEQ == 0

  mesh = plsc.VectorSubcoreMesh(core_axis_name="c", subcore_axis_name="s")

  @functools.partial(
      pl.kernel,
      mesh=mesh,
      compiler_params=pltpu.CompilerParams(use_tc_tiling_on_sc=False),
      out_type=jax.ShapeDtypeStruct((n_rows, EMB), jnp.float32),
      scratch_types=[
          pltpu.VMEM((chunk,), jnp.int32),
          pltpu.VMEM((chunk, EMB), jnp.float32),
          pltpu.VMEM((SEQ, EMB), jnp.float32),
          pltpu.SemaphoreType.DMA,
      ],
  )
  def emb(idx_hbm, table_hbm, pos_hbm, out_hbm, idx_v, rows_v, pos_v, sem):
    wid = lax.axis_index("s") * _NC + lax.axis_index("c")
    pltpu.sync_copy(pos_hbm, pos_v)

    def chunk_body(g, carry):
      base = wid * bpw + g * chunk
      pltpu.sync_copy(idx_hbm.at[pl.ds(base, chunk)], idx_v)
      pltpu.async_copy(table_hbm.at[idx_v], rows_v, sem).wait()

      def row_body(r, c):
        p = lax.rem(r, SEQ)
        for d in range(EMB // _NL):
          sl = pl.ds(d * _NL, _NL)
          rows_v[r, sl] = rows_v[r, sl] * SCALE + pos_v[p, sl]
        return c

      lax.fori_loop(0, chunk, row_body, 0)
      pltpu.sync_copy(rows_v, out_hbm.at[pl.ds(base, chunk)])
      return carry

    lax.fori_loop(0, n_chunks, chunk_body, 0)

  return emb


def kernel(inputs, token_table, position_table):
  b, l = inputs.shape
  n_rows = b * l
  idx = inputs.reshape(n_rows).astype(jnp.int32)
  emb = _build(n_rows, chunk=400)
  out = emb(idx, token_table, position_table)
  return out.reshape(b, l, EMB)


# speedup vs baseline: 1.4391x; 1.4391x over previous
import functools

import jax
import jax.numpy as jnp
from jax import lax
from jax.experimental import pallas as pl
from jax.experimental.pallas import tpu as pltpu
from jax.experimental.pallas import tpu_sc as plsc

SEQ = 200
EMB = 64
SCALE = 8.0

_info = plsc.get_sparse_core_info()
_NC, _NS, _NL = _info.num_cores, _info.num_subcores, _info.num_lanes
_NW = _NC * _NS
_ND = EMB // _NL


def _build(n_rows: int, chunk: int):
  bpw = n_rows // _NW
  n_chunks = bpw // chunk
  seqs = chunk // SEQ
  assert n_rows % (_NW * chunk) == 0
  assert bpw % SEQ == 0
  assert chunk % SEQ == 0
  assert n_chunks >= 4 and n_chunks % 2 == 0

  mesh = plsc.VectorSubcoreMesh(core_axis_name="c", subcore_axis_name="s")

  @functools.partial(
      pl.kernel,
      mesh=mesh,
      compiler_params=pltpu.CompilerParams(use_tc_tiling_on_sc=False),
      out_type=jax.ShapeDtypeStruct((n_rows, EMB), jnp.float32),
      scratch_types=[
          pltpu.VMEM((2, chunk), jnp.int32),
          pltpu.VMEM((chunk, EMB), jnp.float32),
          pltpu.VMEM((chunk, EMB), jnp.float32),
          pltpu.VMEM((SEQ, EMB), jnp.float32),
          pltpu.SemaphoreType.DMA,
          pltpu.SemaphoreType.DMA,
          pltpu.SemaphoreType.DMA,
      ],
  )
  def emb(idx_hbm, table_hbm, pos_hbm, out_hbm,
          idx_v, rows0_v, rows1_v, pos_v, sg, si, so):
    wid = lax.axis_index("s") * _NC + lax.axis_index("c")
    base0 = wid * bpw
    pltpu.sync_copy(pos_hbm, pos_v)
    rows = (rows0_v, rows1_v)

    def idx_copy(g, slot):
      return pltpu.make_async_copy(
          idx_hbm.at[pl.ds(base0 + g * chunk, chunk)], idx_v.at[slot], si)

    def gather(slot):
      return pltpu.make_async_copy(table_hbm.at[idx_v.at[slot]], rows[slot], sg)

    def out_copy(g, slot):
      return pltpu.make_async_copy(
          rows[slot], out_hbm.at[pl.ds(base0 + g * chunk, chunk)], so)

    def compute(buf):
      def body(l, c):
        pv = [pos_v[l, pl.ds(d * _NL, _NL)] for d in range(_ND)]
        for s in range(seqs):
          r = s * SEQ + l
          for d in range(_ND):
            sl = pl.ds(d * _NL, _NL)
            buf[r, sl] = buf[r, sl] * SCALE + pv[d]
        return c

      lax.fori_loop(0, SEQ, body, 0, unroll=2)

    def step(g, slot):
      gather(slot).wait()
      idx_copy(g + 1, 1 - slot).wait()
      out_copy(g - 1, 1 - slot).wait()
      gather(1 - slot).start()

      @pl.when(g + 2 < n_chunks)
      def _():
        idx_copy(g + 2, slot).start()

      compute(rows[slot])
      out_copy(g, slot).start()

    pltpu.sync_copy(idx_hbm.at[pl.ds(base0, chunk)], idx_v.at[0])
    gather(0).start()
    idx_copy(1, 1).start()
    gather(0).wait()
    idx_copy(1, 1).wait()
    gather(1).start()
    idx_copy(2, 0).start()
    compute(rows[0])
    out_copy(0, 0).start()

    def pair(p, c):
      g = 1 + 2 * p
      step(g, 1)
      step(g + 1, 0)
      return c

    lax.fori_loop(0, (n_chunks - 2) // 2, pair, 0)

    gather(1).wait()
    out_copy(n_chunks - 2, 0).wait()
    compute(rows[1])
    out_copy(n_chunks - 1, 1).start()
    out_copy(n_chunks - 1, 1).wait()

  return emb


def kernel(inputs, token_table, position_table):
  b, l = inputs.shape
  n_rows = b * l
  idx = inputs.reshape(n_rows).astype(jnp.int32)
  emb = _build(n_rows, chunk=800)
  out = emb(idx, token_table, position_table)
  return out.reshape(b, l, EMB)


# speedup vs baseline: 1.4403x; 1.0008x over previous
import functools

import jax
import jax.numpy as jnp
from jax import lax
from jax.experimental import pallas as pl
from jax.experimental.pallas import tpu as pltpu
from jax.experimental.pallas import tpu_sc as plsc

SEQ = 200
EMB = 64
SCALE = 8.0

_info = plsc.get_sparse_core_info()
_NC, _NS, _NL = _info.num_cores, _info.num_subcores, _info.num_lanes
_NW = _NC * _NS
_ND = EMB // _NL


def _build(batch: int, cseq: int):
  spw = batch // _NW
  n_chunks = spw // cseq
  assert batch % _NW == 0 and spw % cseq == 0
  assert n_chunks >= 4 and n_chunks % 2 == 0

  mesh = plsc.VectorSubcoreMesh(core_axis_name="c", subcore_axis_name="s")

  @functools.partial(
      pl.kernel,
      mesh=mesh,
      compiler_params=pltpu.CompilerParams(use_tc_tiling_on_sc=False),
      out_type=jax.ShapeDtypeStruct((batch, SEQ, EMB), jnp.float32),
      scratch_types=[
          pltpu.VMEM((2, cseq, SEQ), jnp.int32),
          pltpu.VMEM((cseq, SEQ, EMB), jnp.float32),
          pltpu.VMEM((cseq, SEQ, EMB), jnp.float32),
          pltpu.VMEM((SEQ, EMB), jnp.float32),
          pltpu.SemaphoreType.DMA,
          pltpu.SemaphoreType.DMA,
          pltpu.SemaphoreType.DMA,
      ],
  )
  def emb(idx_hbm, table_hbm, pos_hbm, out_hbm,
          idx_v, rows0_v, rows1_v, pos_v, sg, si, so):
    wid = lax.axis_index("s") * _NC + lax.axis_index("c")
    b0 = wid * spw
    pltpu.sync_copy(pos_hbm, pos_v)
    rows = (rows0_v, rows1_v)

    def idx_copy(g, slot):
      return pltpu.make_async_copy(
          idx_hbm.at[pl.ds(b0 + g * cseq, cseq)], idx_v.at[slot], si)

    def gathers(slot):
      return [
          pltpu.make_async_copy(
              table_hbm.at[idx_v.at[slot, s]], rows[slot].at[s], sg)
          for s in range(cseq)
      ]

    def out_copy(g, slot):
      return pltpu.make_async_copy(
          rows[slot], out_hbm.at[pl.ds(b0 + g * cseq, cseq)], so)

    def start_gathers(slot):
      for c in gathers(slot):
        c.start()

    def wait_gathers(slot):
      for c in gathers(slot):
        c.wait()

    def compute(buf):
      def body(l, c):
        pv = [pos_v[l, pl.ds(d * _NL, _NL)] for d in range(_ND)]
        for s in range(cseq):
          for d in range(_ND):
            sl = pl.ds(d * _NL, _NL)
            buf[s, l, sl] = buf[s, l, sl] * SCALE + pv[d]
        return c

      lax.fori_loop(0, SEQ, body, 0, unroll=2)

    def step(g, slot):
      wait_gathers(slot)
      idx_copy(g + 1, 1 - slot).wait()
      out_copy(g - 1, 1 - slot).wait()
      start_gathers(1 - slot)

      @pl.when(g + 2 < n_chunks)
      def _():
        idx_copy(g + 2, slot).start()

      compute(rows[slot])
      out_copy(g, slot).start()

    pltpu.sync_copy(idx_hbm.at[pl.ds(b0, cseq)], idx_v.at[0])
    start_gathers(0)
    idx_copy(1, 1).start()
    wait_gathers(0)
    idx_copy(1, 1).wait()
    start_gathers(1)
    idx_copy(2, 0).start()
    compute(rows[0])
    out_copy(0, 0).start()

    def pair(p, c):
      g = 1 + 2 * p
      step(g, 1)
      step(g + 1, 0)
      return c

    lax.fori_loop(0, (n_chunks - 2) // 2, pair, 0)

    wait_gathers(1)
    out_copy(n_chunks - 2, 0).wait()
    compute(rows[1])
    out_copy(n_chunks - 1, 1).start()
    out_copy(n_chunks - 1, 1).wait()

  return emb


def kernel(inputs, token_table, position_table):
  batch = inputs.shape[0]
  emb = _build(batch, cseq=4)
  return emb(inputs, token_table, position_table)
